# Initial kernel scaffold; baseline (speedup 1.0000x reference)
import jax
import jax.numpy as jnp
from jax.experimental import pallas as pl


def kernel(x, sample_map):
    raise NotImplementedError("write your pallas kernel here")


# speedup vs baseline: 6.1775x; 6.1775x over previous
import math

import jax
import jax.numpy as jnp
from jax import lax
from jax.experimental import pallas as pl
from jax.experimental.pallas import tpu as pltpu
from jax.experimental.pallas import tpu_sc as plsc

_L = 16
_NW = 32
_S = 128
_GW = 8


def _resample_sc(tbl, u, v, P, N, H, W, Wp, CH):
    n_per_w = N // _NW
    n_chunks = n_per_w // CH
    G = CH // _S
    f32 = jnp.float32
    i32 = jnp.int32
    nA = (H * Wp) // _GW
    Wp8 = Wp // _GW

    mesh = plsc.VectorSubcoreMesh(core_axis_name="c", subcore_axis_name="s")

    def body(tbl_hbm, u_hbm, v_hbm, out_hbm,
             u_v, v_v, o_v, i0_m, i1_m, g0, g1, out_v, s0, s1):
        cid = lax.axis_index("c")
        sid = lax.axis_index("s")
        w = sid * 2 + cid
        base_w = w * n_per_w

        lanes = lax.iota(i32, _L)

        def chunk_body(ci, carry):
            base = base_w + ci * CH
            pltpu.sync_copy(u_hbm.at[pl.ds(base, CH)], u_v)
            pltpu.sync_copy(v_hbm.at[pl.ds(base, CH)], v_v)

            def idx_body(i, c2):
                sl = pl.ds(i * _L, _L)
                gi = jnp.right_shift(i, 3)
                oi = pl.ds(jnp.bitwise_and(i, 7) * _L, _L)
                uu = u_v[sl]
                vv = v_v[sl]
                xt = uu.astype(i32)
                x0i = jnp.where(uu < xt.astype(f32), xt - 1, xt)
                u_v[sl] = uu - x0i.astype(f32)
                yt = vv.astype(i32)
                y0i = jnp.where(vv < yt.astype(f32), yt - 1, yt)
                v_v[sl] = vv - y0i.astype(f32)
                x0 = jnp.bitwise_and(x0i, W - 1)
                y0 = jnp.minimum(jnp.maximum(y0i, 0), H - 1)
                y1 = jnp.minimum(y0 + 1, H - 1)
                off = jnp.bitwise_and(x0, _GW - 1)
                last = off == _GW - 1
                o_v[sl] = jnp.where(last, 3, off)
                voff = jnp.where(last, nA, 0)
                xb = jnp.right_shift(x0, 3)
                i0_m[gi, oi] = y0 * Wp8 + xb + voff
                i1_m[gi, oi] = y1 * Wp8 + xb + voff
                return c2

            lax.fori_loop(0, CH // _L, idx_body, 0)

            def plane_body(p, c2):
                pv = tbl_hbm.at[p]

                def fire_body(g, c3):
                    pltpu.async_copy(pv.at[i0_m.at[g]], g0.at[g], s0)
                    pltpu.async_copy(pv.at[i1_m.at[g]], g1.at[g], s1)
                    return c3

                lax.fori_loop(0, G, fire_body, 0)

                def drain_body(g, c3):
                    pltpu.make_async_copy(
                        pv.at[i0_m.at[g]], g0.at[g], s0).wait()
                    pltpu.make_async_copy(
                        pv.at[i1_m.at[g]], g1.at[g], s1).wait()
                    return c3

                lax.fori_loop(0, G, drain_body, 0)

                def comb_body(i, c4):
                    g = jnp.right_shift(i, 3)
                    pos = lanes + jnp.bitwise_and(i, 7) * _L
                    sl = pl.ds(i * _L, _L)
                    wx = u_v[sl]
                    wy = v_v[sl]
                    o0 = o_v[sl]
                    o1 = o0 + 1
                    p00 = plsc.load_gather(g0.at[g], [pos, o0])
                    p01 = plsc.load_gather(g0.at[g], [pos, o1])
                    p10 = plsc.load_gather(g1.at[g], [pos, o0])
                    p11 = plsc.load_gather(g1.at[g], [pos, o1])
                    top = p00 + wx * (p01 - p00)
                    bot = p10 + wx * (p11 - p10)
                    out_v[sl] = top + wy * (bot - top)
                    return c4

                lax.fori_loop(0, CH // _L, comb_body, 0)
                pltpu.sync_copy(out_v, out_hbm.at[p, pl.ds(base, CH)])
                return c2

            lax.fori_loop(0, P, plane_body, 0)
            return carry

        lax.fori_loop(0, n_chunks, chunk_body, 0)

    call = pl.kernel(
        body,
        out_type=jax.ShapeDtypeStruct((P, N), f32),
        mesh=mesh,
        compiler_params=pltpu.CompilerParams(
            needs_layout_passes=False, use_tc_tiling_on_sc=False),
        scratch_types=[
            pltpu.VMEM((CH,), f32),
            pltpu.VMEM((CH,), f32),
            pltpu.VMEM((CH,), i32),
            pltpu.VMEM((CH // _S, _S), i32),
            pltpu.VMEM((CH // _S, _S), i32),
            pltpu.VMEM((CH // _S, _S, _GW), f32),
            pltpu.VMEM((CH // _S, _S, _GW), f32),
            pltpu.VMEM((CH,), f32),
            pltpu.SemaphoreType.DMA,
            pltpu.SemaphoreType.DMA,
        ],
    )
    return call(tbl, u, v)


def kernel(x, sample_map):
    B, C, H, W = x.shape
    F, K2 = sample_map.shape[0], sample_map.shape[1]
    N = F * K2
    P = B * C
    g = int(math.isqrt(K2))
    assert (W & (W - 1)) == 0, "W must be a power of two"
    assert N % _NW == 0
    CH = 4096
    assert (N // _NW) % CH == 0

    Wp = W + _GW
    nA = (H * Wp) // _GW
    padded = jnp.concatenate([x, x[..., :_GW]], axis=-1)
    flat = padded.reshape(P, H * Wp)
    tbl_a = flat.reshape(P, nA, _GW)
    tbl_b = flat[:, 4:-4].reshape(P, nA - 1, _GW)
    pad_row = jnp.zeros((P, 1, _GW), x.dtype)
    tbl = jnp.concatenate([tbl_a, tbl_b, pad_row], axis=1)

    u = sample_map[..., 0].reshape(N)
    v = sample_map[..., 1].reshape(N)
    out = _resample_sc(tbl, u, v, P, N, H, W, Wp, CH)
    return out.reshape(B, C, F, g, g)


# speedup vs baseline: 6.3001x; 1.0198x over previous
import math

import jax
import jax.numpy as jnp
from jax import lax
from jax.experimental import pallas as pl
from jax.experimental.pallas import tpu as pltpu
from jax.experimental.pallas import tpu_sc as plsc

_L = 16
_NW = 32
_S = 128
_GW = 8


def _resample_sc(tbl, u, v, P, N, H, W, Wp, CH):
    n_per_w = N // _NW
    n_chunks = n_per_w // CH
    G = CH // _S
    f32 = jnp.float32
    i32 = jnp.int32
    nA = (H * Wp) // _GW
    Wp8 = Wp // _GW

    mesh = plsc.VectorSubcoreMesh(core_axis_name="c", subcore_axis_name="s")

    def body(tbl_hbm, u_hbm, v_hbm, out_hbm,
             u_v, v_v, o_v, i0_m, i1_m, g0, g1, out_v, s00, s01, s10, s11):
        sems = ((s00, s01), (s10, s11))
        cid = lax.axis_index("c")
        sid = lax.axis_index("s")
        w = sid * 2 + cid
        base_w = w * n_per_w

        lanes = lax.iota(i32, _L)

        def chunk_body(ci, carry):
            base = base_w + ci * CH
            pltpu.sync_copy(u_hbm.at[pl.ds(base, CH)], u_v)
            pltpu.sync_copy(v_hbm.at[pl.ds(base, CH)], v_v)

            def idx_body(i, c2):
                sl = pl.ds(i * _L, _L)
                gi = jnp.right_shift(i, 3)
                oi = pl.ds(jnp.bitwise_and(i, 7) * _L, _L)
                uu = u_v[sl]
                vv = v_v[sl]
                xt = uu.astype(i32)
                x0i = jnp.where(uu < xt.astype(f32), xt - 1, xt)
                u_v[sl] = uu - x0i.astype(f32)
                yt = vv.astype(i32)
                y0i = jnp.where(vv < yt.astype(f32), yt - 1, yt)
                v_v[sl] = vv - y0i.astype(f32)
                x0 = jnp.bitwise_and(x0i, W - 1)
                y0 = jnp.minimum(jnp.maximum(y0i, 0), H - 1)
                y1 = jnp.minimum(y0 + 1, H - 1)
                off = jnp.bitwise_and(x0, _GW - 1)
                last = off == _GW - 1
                o_v[sl] = jnp.where(last, 3, off)
                voff = jnp.where(last, nA, 0)
                xb = jnp.right_shift(x0, 3)
                i0_m[gi, oi] = y0 * Wp8 + xb + voff
                i1_m[gi, oi] = y1 * Wp8 + xb + voff
                return c2

            lax.fori_loop(0, CH // _L, idx_body, 0)

            def fire(p, b):
                pv = tbl_hbm.at[p]
                ta, tb = sems[b]

                def fire_body(g, c3):
                    pltpu.async_copy(pv.at[i0_m.at[g]], g0.at[b, g], ta)
                    pltpu.async_copy(pv.at[i1_m.at[g]], g1.at[b, g], tb)
                    return c3

                lax.fori_loop(0, G, fire_body, 0)

            def process(p, b):
                pv = tbl_hbm.at[p]
                ta, tb = sems[b]

                def drain_body(g, c3):
                    pltpu.make_async_copy(
                        pv.at[i0_m.at[g]], g0.at[b, g], ta).wait()
                    pltpu.make_async_copy(
                        pv.at[i1_m.at[g]], g1.at[b, g], tb).wait()
                    return c3

                lax.fori_loop(0, G, drain_body, 0)

                def comb_body(i, c4):
                    g = jnp.right_shift(i, 3)
                    pos = lanes + jnp.bitwise_and(i, 7) * _L
                    sl = pl.ds(i * _L, _L)
                    wx = u_v[sl]
                    wy = v_v[sl]
                    o0 = o_v[sl]
                    o1 = o0 + 1
                    p00 = plsc.load_gather(g0.at[b, g], [pos, o0])
                    p01 = plsc.load_gather(g0.at[b, g], [pos, o1])
                    p10 = plsc.load_gather(g1.at[b, g], [pos, o0])
                    p11 = plsc.load_gather(g1.at[b, g], [pos, o1])
                    top = p00 + wx * (p01 - p00)
                    bot = p10 + wx * (p11 - p10)
                    out_v[sl] = top + wy * (bot - top)
                    return c4

                lax.fori_loop(0, CH // _L, comb_body, 0)
                pltpu.sync_copy(out_v, out_hbm.at[p, pl.ds(base, CH)])

            fire(0, 0)
            for p in range(P):
                if p + 1 < P:
                    fire(p + 1, (p + 1) % 2)
                process(p, p % 2)
            return carry

        lax.fori_loop(0, n_chunks, chunk_body, 0)

    call = pl.kernel(
        body,
        out_type=jax.ShapeDtypeStruct((P, N), f32),
        mesh=mesh,
        compiler_params=pltpu.CompilerParams(
            needs_layout_passes=False, use_tc_tiling_on_sc=False),
        scratch_types=[
            pltpu.VMEM((CH,), f32),
            pltpu.VMEM((CH,), f32),
            pltpu.VMEM((CH,), i32),
            pltpu.VMEM((CH // _S, _S), i32),
            pltpu.VMEM((CH // _S, _S), i32),
            pltpu.VMEM((2, CH // _S, _S, _GW), f32),
            pltpu.VMEM((2, CH // _S, _S, _GW), f32),
            pltpu.VMEM((CH,), f32),
            pltpu.SemaphoreType.DMA,
            pltpu.SemaphoreType.DMA,
            pltpu.SemaphoreType.DMA,
            pltpu.SemaphoreType.DMA,
        ],
    )
    return call(tbl, u, v)


def kernel(x, sample_map):
    B, C, H, W = x.shape
    F, K2 = sample_map.shape[0], sample_map.shape[1]
    N = F * K2
    P = B * C
    g = int(math.isqrt(K2))
    assert (W & (W - 1)) == 0, "W must be a power of two"
    assert N % _NW == 0
    CH = 2048
    assert (N // _NW) % CH == 0

    Wp = W + _GW
    nA = (H * Wp) // _GW
    padded = jnp.concatenate([x, x[..., :_GW]], axis=-1)
    flat = padded.reshape(P, H * Wp)
    tbl_a = flat.reshape(P, nA, _GW)
    tbl_b = flat[:, 4:-4].reshape(P, nA - 1, _GW)
    pad_row = jnp.zeros((P, 1, _GW), x.dtype)
    tbl = jnp.concatenate([tbl_a, tbl_b, pad_row], axis=1)

    u = sample_map[..., 0].reshape(N)
    v = sample_map[..., 1].reshape(N)
    out = _resample_sc(tbl, u, v, P, N, H, W, Wp, CH)
    return out.reshape(B, C, F, g, g)


# speedup vs baseline: 44.9190x; 7.1298x over previous
import math

import jax
import jax.numpy as jnp
from jax import lax
from jax.experimental import pallas as pl
from jax.experimental.pallas import tpu as pltpu
from jax.experimental.pallas import tpu_sc as plsc

_L = 16
_NW = 32
_S = 128
_GW = 8


def _resample_sc(img, u, v, P, N, H, W, CH):
    n_per_w = N // _NW
    n_chunks = n_per_w // CH
    G = CH // _S
    f32 = jnp.float32
    i32 = jnp.int32
    W8 = W // _GW

    mesh = plsc.VectorSubcoreMesh(core_axis_name="c", subcore_axis_name="s")

    def body(img_hbm, u_hbm, v_hbm, out_hbm,
             u_v, v_v, ol_v, or_v, il0, ir0, il1, ir1,
             gl0, gr0, gl1, gr1, out_v, s0, s1, s2, s3):
        cid = lax.axis_index("c")
        sid = lax.axis_index("s")
        w = sid * 2 + cid
        base_w = w * n_per_w

        lanes = lax.iota(i32, _L)

        def chunk_body(ci, carry):
            base = base_w + ci * CH
            pltpu.sync_copy(u_hbm.at[pl.ds(base, CH)], u_v)
            pltpu.sync_copy(v_hbm.at[pl.ds(base, CH)], v_v)

            def idx_body(i, c2):
                sl = pl.ds(i * _L, _L)
                gi = jnp.right_shift(i, 3)
                oi = pl.ds(jnp.bitwise_and(i, 7) * _L, _L)
                uu = u_v[sl]
                vv = v_v[sl]
                xt = uu.astype(i32)
                x0i = jnp.where(uu < xt.astype(f32), xt - 1, xt)
                u_v[sl] = uu - x0i.astype(f32)
                yt = vv.astype(i32)
                y0i = jnp.where(vv < yt.astype(f32), yt - 1, yt)
                v_v[sl] = vv - y0i.astype(f32)
                x0 = jnp.bitwise_and(x0i, W - 1)
                x1 = jnp.bitwise_and(x0 + 1, W - 1)
                y0 = jnp.minimum(jnp.maximum(y0i, 0), H - 1)
                y1 = jnp.minimum(y0 + 1, H - 1)
                ol_v[sl] = jnp.bitwise_and(x0, _GW - 1)
                or_v[sl] = jnp.bitwise_and(x1, _GW - 1)
                xbl = jnp.right_shift(x0, 3)
                xbr = jnp.right_shift(x1, 3)
                r0 = y0 * W8
                r1 = y1 * W8
                il0[gi, oi] = r0 + xbl
                ir0[gi, oi] = r0 + xbr
                il1[gi, oi] = r1 + xbl
                ir1[gi, oi] = r1 + xbr
                return c2

            lax.fori_loop(0, CH // _L, idx_body, 0)

            def plane_body(p, c2):
                pv = img_hbm.at[p]

                def fire_body(g, c3):
                    pltpu.async_copy(pv.at[il0.at[g]], gl0.at[g], s0)
                    pltpu.async_copy(pv.at[ir0.at[g]], gr0.at[g], s1)
                    pltpu.async_copy(pv.at[il1.at[g]], gl1.at[g], s2)
                    pltpu.async_copy(pv.at[ir1.at[g]], gr1.at[g], s3)
                    return c3

                lax.fori_loop(0, G, fire_body, 0)

                def drain_body(g, c3):
                    pltpu.make_async_copy(pv.at[il0.at[g]], gl0.at[g], s0).wait()
                    pltpu.make_async_copy(pv.at[ir0.at[g]], gr0.at[g], s1).wait()
                    pltpu.make_async_copy(pv.at[il1.at[g]], gl1.at[g], s2).wait()
                    pltpu.make_async_copy(pv.at[ir1.at[g]], gr1.at[g], s3).wait()
                    return c3

                lax.fori_loop(0, G, drain_body, 0)

                def comb_body(i, c4):
                    g = jnp.right_shift(i, 3)
                    pos = lanes + jnp.bitwise_and(i, 7) * _L
                    sl = pl.ds(i * _L, _L)
                    wx = u_v[sl]
                    wy = v_v[sl]
                    o0 = ol_v[sl]
                    o1 = or_v[sl]
                    p00 = plsc.load_gather(gl0.at[g], [pos, o0])
                    p01 = plsc.load_gather(gr0.at[g], [pos, o1])
                    p10 = plsc.load_gather(gl1.at[g], [pos, o0])
                    p11 = plsc.load_gather(gr1.at[g], [pos, o1])
                    top = p00 + wx * (p01 - p00)
                    bot = p10 + wx * (p11 - p10)
                    out_v[sl] = top + wy * (bot - top)
                    return c4

                lax.fori_loop(0, CH // _L, comb_body, 0)
                pltpu.sync_copy(out_v, out_hbm.at[p, pl.ds(base, CH)])
                return c2

            lax.fori_loop(0, P, plane_body, 0)
            return carry

        lax.fori_loop(0, n_chunks, chunk_body, 0)

    call = pl.kernel(
        body,
        out_type=jax.ShapeDtypeStruct((P, N), f32),
        mesh=mesh,
        compiler_params=pltpu.CompilerParams(
            needs_layout_passes=False, use_tc_tiling_on_sc=False),
        scratch_types=[
            pltpu.VMEM((CH,), f32),
            pltpu.VMEM((CH,), f32),
            pltpu.VMEM((CH,), i32),
            pltpu.VMEM((CH,), i32),
            pltpu.VMEM((CH // _S, _S), i32),
            pltpu.VMEM((CH // _S, _S), i32),
            pltpu.VMEM((CH // _S, _S), i32),
            pltpu.VMEM((CH // _S, _S), i32),
            pltpu.VMEM((CH // _S, _S, _GW), f32),
            pltpu.VMEM((CH // _S, _S, _GW), f32),
            pltpu.VMEM((CH // _S, _S, _GW), f32),
            pltpu.VMEM((CH // _S, _S, _GW), f32),
            pltpu.VMEM((CH,), f32),
            pltpu.SemaphoreType.DMA,
            pltpu.SemaphoreType.DMA,
            pltpu.SemaphoreType.DMA,
            pltpu.SemaphoreType.DMA,
        ],
    )
    return call(img, u, v)


def kernel(x, sample_map):
    B, C, H, W = x.shape
    F, K2 = sample_map.shape[0], sample_map.shape[1]
    N = F * K2
    P = B * C
    g = int(math.isqrt(K2))
    assert (W & (W - 1)) == 0, "W must be a power of two"
    assert W % _GW == 0
    assert N % _NW == 0
    CH = 2048
    assert (N // _NW) % CH == 0

    img = x.reshape(P, (H * W) // _GW, _GW)
    u = sample_map[..., 0].reshape(N)
    v = sample_map[..., 1].reshape(N)
    out = _resample_sc(img, u, v, P, N, H, W, CH)
    return out.reshape(B, C, F, g, g)


# speedup vs baseline: 48.3184x; 1.0757x over previous
import math

import jax
import jax.numpy as jnp
from jax import lax
from jax.experimental import pallas as pl
from jax.experimental.pallas import tpu as pltpu
from jax.experimental.pallas import tpu_sc as plsc

_L = 16
_NW = 32
_S = 128
_GW = 8


def _resample_sc(img, u, v, P, N, H, W, CH):
    n_per_w = N // _NW
    n_chunks = n_per_w // CH
    G = CH // _S
    f32 = jnp.float32
    i32 = jnp.int32
    W8 = W // _GW

    mesh = plsc.VectorSubcoreMesh(core_axis_name="c", subcore_axis_name="s")

    def body(img_hbm, u_hbm, v_hbm, out_hbm,
             u_v, v_v, ol_v, or_v, il0, ir0, il1, ir1,
             gl0, gr0, gl1, gr1, out_v,
             s00, s01, s02, s03, s10, s11, s12, s13):
        sems = ((s00, s01, s02, s03), (s10, s11, s12, s13))
        cid = lax.axis_index("c")
        sid = lax.axis_index("s")
        w = sid * 2 + cid
        base_w = w * n_per_w

        lanes = lax.iota(i32, _L)

        def chunk_body(ci, carry):
            base = base_w + ci * CH
            pltpu.sync_copy(u_hbm.at[pl.ds(base, CH)], u_v)
            pltpu.sync_copy(v_hbm.at[pl.ds(base, CH)], v_v)

            def idx_body(i, c2):
                sl = pl.ds(i * _L, _L)
                gi = jnp.right_shift(i, 3)
                oi = pl.ds(jnp.bitwise_and(i, 7) * _L, _L)
                uu = u_v[sl]
                vv = v_v[sl]
                xt = uu.astype(i32)
                x0i = jnp.where(uu < xt.astype(f32), xt - 1, xt)
                u_v[sl] = uu - x0i.astype(f32)
                yt = vv.astype(i32)
                y0i = jnp.where(vv < yt.astype(f32), yt - 1, yt)
                v_v[sl] = vv - y0i.astype(f32)
                x0 = jnp.bitwise_and(x0i, W - 1)
                x1 = jnp.bitwise_and(x0 + 1, W - 1)
                y0 = jnp.minimum(jnp.maximum(y0i, 0), H - 1)
                y1 = jnp.minimum(y0 + 1, H - 1)
                ol_v[sl] = jnp.bitwise_and(x0, _GW - 1)
                or_v[sl] = jnp.bitwise_and(x1, _GW - 1)
                xbl = jnp.right_shift(x0, 3)
                xbr = jnp.right_shift(x1, 3)
                r0 = y0 * W8
                r1 = y1 * W8
                il0[gi, oi] = r0 + xbl
                ir0[gi, oi] = r0 + xbr
                il1[gi, oi] = r1 + xbl
                ir1[gi, oi] = r1 + xbr
                return c2

            lax.fori_loop(0, CH // _L, idx_body, 0)

            def fire(p, b):
                pv = img_hbm.at[p]
                t0, t1, t2, t3 = sems[b]

                def fire_body(g, c3):
                    pltpu.async_copy(pv.at[il0.at[g]], gl0.at[b, g], t0)
                    pltpu.async_copy(pv.at[ir0.at[g]], gr0.at[b, g], t1)
                    pltpu.async_copy(pv.at[il1.at[g]], gl1.at[b, g], t2)
                    pltpu.async_copy(pv.at[ir1.at[g]], gr1.at[b, g], t3)
                    return c3

                lax.fori_loop(0, G, fire_body, 0)

            def process(p, b):
                pv = img_hbm.at[p]
                t0, t1, t2, t3 = sems[b]

                def drain_body(g, c3):
                    pltpu.make_async_copy(pv.at[il0.at[g]], gl0.at[b, g], t0).wait()
                    pltpu.make_async_copy(pv.at[ir0.at[g]], gr0.at[b, g], t1).wait()
                    pltpu.make_async_copy(pv.at[il1.at[g]], gl1.at[b, g], t2).wait()
                    pltpu.make_async_copy(pv.at[ir1.at[g]], gr1.at[b, g], t3).wait()
                    return c3

                lax.fori_loop(0, G, drain_body, 0)

                def comb_body(i, c4):
                    g = jnp.right_shift(i, 3)
                    pos = lanes + jnp.bitwise_and(i, 7) * _L
                    sl = pl.ds(i * _L, _L)
                    wx = u_v[sl]
                    wy = v_v[sl]
                    o0 = ol_v[sl]
                    o1 = or_v[sl]
                    p00 = plsc.load_gather(gl0.at[b, g], [pos, o0])
                    p01 = plsc.load_gather(gr0.at[b, g], [pos, o1])
                    p10 = plsc.load_gather(gl1.at[b, g], [pos, o0])
                    p11 = plsc.load_gather(gr1.at[b, g], [pos, o1])
                    top = p00 + wx * (p01 - p00)
                    bot = p10 + wx * (p11 - p10)
                    out_v[sl] = top + wy * (bot - top)
                    return c4

                lax.fori_loop(0, CH // _L, comb_body, 0)
                pltpu.sync_copy(out_v, out_hbm.at[p, pl.ds(base, CH)])

            fire(0, 0)
            for p in range(P):
                if p + 1 < P:
                    fire(p + 1, (p + 1) % 2)
                process(p, p % 2)
            return carry

        lax.fori_loop(0, n_chunks, chunk_body, 0)

    call = pl.kernel(
        body,
        out_type=jax.ShapeDtypeStruct((P, N), f32),
        mesh=mesh,
        compiler_params=pltpu.CompilerParams(
            needs_layout_passes=False, use_tc_tiling_on_sc=False),
        scratch_types=[
            pltpu.VMEM((CH,), f32),
            pltpu.VMEM((CH,), f32),
            pltpu.VMEM((CH,), i32),
            pltpu.VMEM((CH,), i32),
            pltpu.VMEM((CH // _S, _S), i32),
            pltpu.VMEM((CH // _S, _S), i32),
            pltpu.VMEM((CH // _S, _S), i32),
            pltpu.VMEM((CH // _S, _S), i32),
            pltpu.VMEM((2, CH // _S, _S, _GW), f32),
            pltpu.VMEM((2, CH // _S, _S, _GW), f32),
            pltpu.VMEM((2, CH // _S, _S, _GW), f32),
            pltpu.VMEM((2, CH // _S, _S, _GW), f32),
            pltpu.VMEM((CH,), f32),
            pltpu.SemaphoreType.DMA,
            pltpu.SemaphoreType.DMA,
            pltpu.SemaphoreType.DMA,
            pltpu.SemaphoreType.DMA,
            pltpu.SemaphoreType.DMA,
            pltpu.SemaphoreType.DMA,
            pltpu.SemaphoreType.DMA,
            pltpu.SemaphoreType.DMA,
        ],
    )
    return call(img, u, v)


def kernel(x, sample_map):
    B, C, H, W = x.shape
    F, K2 = sample_map.shape[0], sample_map.shape[1]
    N = F * K2
    P = B * C
    g = int(math.isqrt(K2))
    assert (W & (W - 1)) == 0, "W must be a power of two"
    assert W % _GW == 0
    assert N % _NW == 0
    CH = 1024
    assert (N // _NW) % CH == 0

    img = x.reshape(P, (H * W) // _GW, _GW)
    u = sample_map[..., 0].reshape(N)
    v = sample_map[..., 1].reshape(N)
    out = _resample_sc(img, u, v, P, N, H, W, CH)
    return out.reshape(B, C, F, g, g)
